# baseline (speedup 1.0000x reference)
import functools

import jax
import jax.numpy as jnp
from jax.experimental import pallas as pl
from jax.experimental.pallas import tpu as pltpu
from jax.experimental.pallas import tpu_sc as plsc

_NUM_ENTITIES = 1000000
_EMBED = 64
_BATCH = 16384
_H1, _H2 = 512, 256
_FEAT = 7 * _EMBED

_NUM_IDX = 3 * _BATCH
_GATHER_WINDOW = 128
_BLOCK_B = 2048


_SC_CORES = 2
_SC_SUBCORES = 16
_NW = _SC_CORES * _SC_SUBCORES
_B_PER_W = _NUM_IDX // _NW


_WIN = 128
_WIDE = 2 * _EMBED
_HALF = _NUM_ENTITIES // 2
_PACK_R = 4000


def _pack_kernel(t_ref, o_ref):
    h = pl.program_id(1)

    @pl.when(h == 0)
    def _lo():
        o_ref[:, :_EMBED] = t_ref[...]

    @pl.when(h == 1)
    def _hi():
        o_ref[:, _EMBED:] = t_ref[...]


def _tc_pack(table):
    n_row_blocks = _HALF // _PACK_R
    return pl.pallas_call(
        _pack_kernel,
        grid=(n_row_blocks, 2),
        in_specs=[
            pl.BlockSpec((_PACK_R, _EMBED), lambda j, h: (h * n_row_blocks + j, 0))
        ],
        out_specs=pl.BlockSpec((_PACK_R, _WIDE), lambda j, h: (j, 0)),
        out_shape=jax.ShapeDtypeStruct((_HALF, _WIDE), jnp.float32),
        compiler_params=pltpu.CompilerParams(
            dimension_semantics=("parallel", "arbitrary")
        ),
    )(table)


def _sc_gather_wide(tablew, idxw):
    mesh = plsc.VectorSubcoreMesh(core_axis_name="c", subcore_axis_name="s")

    @functools.partial(
        pl.kernel,
        mesh=mesh,
        out_type=jax.ShapeDtypeStruct((_NUM_IDX, _WIDE), jnp.float32),
    )
    def gather_kernel(tbl_hbm, idx_hbm, out_hbm):
        def body(i_vmem, o_vmem):
            pltpu.sync_copy(tbl_hbm.at[i_vmem.at[0]], o_vmem)

        pltpu.emit_pipeline(
            body,
            grid=(_NUM_IDX // _WIN,),
            in_specs=[pl.BlockSpec((1, _WIN), lambda i: (0, i))],
            out_specs=[pl.BlockSpec((_WIN, _WIDE), lambda i: (i, 0))],
            core_axis_name=("c", "s"),
            dimension_semantics=(pltpu.PARALLEL,),
        )(idx_hbm, out_hbm)

    return gather_kernel(tablew, idxw.reshape(1, _NUM_IDX))


def _mlp_kernel(
    g_ref, p_ref, w1_ref, b1_ref, w2_ref, b2_ref, wout_ref, bout_ref, o_ref
):
    def pick(k):
        g = g_ref[k]
        p = p_ref[k]
        return jnp.where(p == 0, g[:, :_EMBED], g[:, _EMBED:])

    e1 = pick(0)
    e2 = pick(1)
    et = pick(2)
    feats = jnp.concatenate(
        [e1, e2, et, e1 * e2, e1 * et, e2 * et, e1 - e2], axis=-1
    )
    h = jnp.dot(feats, w1_ref[...], preferred_element_type=jnp.float32)
    h = jnp.maximum(h + b1_ref[...], 0.0)
    h = jnp.dot(h, w2_ref[...], preferred_element_type=jnp.float32)
    h = jnp.maximum(h + b2_ref[...], 0.0)
    out = jnp.dot(h, wout_ref[...], preferred_element_type=jnp.float32)
    o_ref[...] = out + bout_ref[0, 0]


def kernel(c1, c2, target, table, W1, b1, W2, b2, Wout, bout):
    idx = jnp.concatenate([c1, c2, target], axis=0).astype(jnp.int32)
    idxw = jnp.where(idx < _HALF, idx, idx - _HALF)
    par = (idx >= _HALF).astype(jnp.int32).reshape(3, _BATCH, 1)
    tablew = _tc_pack(table)
    g = _sc_gather_wide(tablew, idxw).reshape(3, _BATCH, _WIDE)

    out = pl.pallas_call(
        _mlp_kernel,
        grid=(_BATCH // _BLOCK_B,),
        in_specs=[
            pl.BlockSpec((3, _BLOCK_B, _WIDE), lambda i: (0, i, 0)),
            pl.BlockSpec((3, _BLOCK_B, 1), lambda i: (0, i, 0)),
            pl.BlockSpec((_FEAT, _H1), lambda i: (0, 0)),
            pl.BlockSpec((1, _H1), lambda i: (0, 0)),
            pl.BlockSpec((_H1, _H2), lambda i: (0, 0)),
            pl.BlockSpec((1, _H2), lambda i: (0, 0)),
            pl.BlockSpec((_H2, 1), lambda i: (0, 0)),
            pl.BlockSpec((1, 1), lambda i: (0, 0)),
        ],
        out_specs=pl.BlockSpec((_BLOCK_B, 1), lambda i: (i, 0)),
        out_shape=jax.ShapeDtypeStruct((_BATCH, 1), jnp.float32),
    )(
        g,
        par,
        W1,
        b1.reshape(1, _H1),
        W2,
        b2.reshape(1, _H2),
        Wout,
        bout.reshape(1, 1),
    )
    return out[:, 0]


# speedup vs baseline: 1.1637x; 1.1637x over previous
import functools

import jax
import jax.numpy as jnp
from jax.experimental import pallas as pl
from jax.experimental.pallas import tpu as pltpu
from jax.experimental.pallas import tpu_sc as plsc

_NUM_ENTITIES = 1000000
_EMBED = 64
_BATCH = 16384
_H1, _H2 = 512, 256
_FEAT = 7 * _EMBED

_NUM_IDX = 3 * _BATCH
_GATHER_WINDOW = 128
_BLOCK_B = 2048


_SC_CORES = 2
_SC_SUBCORES = 16
_NW = _SC_CORES * _SC_SUBCORES
_B_PER_W = _NUM_IDX // _NW


_WIN = 128
_WIDE = 2 * _EMBED
_HALF = _NUM_ENTITIES // 2
_PACK_R = 4000


def _pack_kernel(t_ref, o_ref):
    o_ref[:, :_EMBED] = t_ref[:_PACK_R, :]
    o_ref[:, _EMBED:] = t_ref[_PACK_R:, :]


def _tc_pack(table):
    return pl.pallas_call(
        _pack_kernel,
        grid=(_NUM_ENTITIES // (2 * _PACK_R),),
        in_specs=[pl.BlockSpec((2 * _PACK_R, _EMBED), lambda i: (i, 0))],
        out_specs=pl.BlockSpec((_PACK_R, _WIDE), lambda i: (i, 0)),
        out_shape=jax.ShapeDtypeStruct((_HALF, _WIDE), jnp.float32),
        compiler_params=pltpu.CompilerParams(
            dimension_semantics=("parallel",)
        ),
    )(table)


def _sc_gather_wide(tablew, idxw):
    mesh = plsc.VectorSubcoreMesh(core_axis_name="c", subcore_axis_name="s")

    @functools.partial(
        pl.kernel,
        mesh=mesh,
        out_type=jax.ShapeDtypeStruct((_NUM_IDX, _WIDE), jnp.float32),
    )
    def gather_kernel(tbl_hbm, idx_hbm, out_hbm):
        def body(i_vmem, o_vmem):
            pltpu.sync_copy(tbl_hbm.at[i_vmem.at[0]], o_vmem)

        pltpu.emit_pipeline(
            body,
            grid=(_NUM_IDX // _WIN,),
            in_specs=[pl.BlockSpec((1, _WIN), lambda i: (0, i))],
            out_specs=[pl.BlockSpec((_WIN, _WIDE), lambda i: (i, 0))],
            core_axis_name=("c", "s"),
            dimension_semantics=(pltpu.PARALLEL,),
        )(idx_hbm, out_hbm)

    return gather_kernel(tablew, idxw.reshape(1, _NUM_IDX))


def _mlp_kernel(
    g_ref, p_ref, w1_ref, b1_ref, w2_ref, b2_ref, wout_ref, bout_ref, o_ref
):
    def pick(k):
        g = g_ref[k]
        p = p_ref[k]
        return jnp.where(p == 0, g[:, :_EMBED], g[:, _EMBED:])

    e1 = pick(0)
    e2 = pick(1)
    et = pick(2)
    feats = jnp.concatenate(
        [e1, e2, et, e1 * e2, e1 * et, e2 * et, e1 - e2], axis=-1
    )
    h = jnp.dot(feats, w1_ref[...], preferred_element_type=jnp.float32)
    h = jnp.maximum(h + b1_ref[...], 0.0)
    h = jnp.dot(h, w2_ref[...], preferred_element_type=jnp.float32)
    h = jnp.maximum(h + b2_ref[...], 0.0)
    out = jnp.dot(h, wout_ref[...], preferred_element_type=jnp.float32)
    o_ref[...] = out + bout_ref[0, 0]


def kernel(c1, c2, target, table, W1, b1, W2, b2, Wout, bout):
    idx = jnp.concatenate([c1, c2, target], axis=0).astype(jnp.int32)
    blk = idx // (2 * _PACK_R)
    rem = idx % (2 * _PACK_R)
    idxw = blk * _PACK_R + jnp.where(rem < _PACK_R, rem, rem - _PACK_R)
    par = (rem >= _PACK_R).astype(jnp.int32).reshape(3, _BATCH, 1)
    tablew = _tc_pack(table)
    g = _sc_gather_wide(tablew, idxw).reshape(3, _BATCH, _WIDE)

    out = pl.pallas_call(
        _mlp_kernel,
        grid=(_BATCH // _BLOCK_B,),
        in_specs=[
            pl.BlockSpec((3, _BLOCK_B, _WIDE), lambda i: (0, i, 0)),
            pl.BlockSpec((3, _BLOCK_B, 1), lambda i: (0, i, 0)),
            pl.BlockSpec((_FEAT, _H1), lambda i: (0, 0)),
            pl.BlockSpec((1, _H1), lambda i: (0, 0)),
            pl.BlockSpec((_H1, _H2), lambda i: (0, 0)),
            pl.BlockSpec((1, _H2), lambda i: (0, 0)),
            pl.BlockSpec((_H2, 1), lambda i: (0, 0)),
            pl.BlockSpec((1, 1), lambda i: (0, 0)),
        ],
        out_specs=pl.BlockSpec((_BLOCK_B, 1), lambda i: (i, 0)),
        out_shape=jax.ShapeDtypeStruct((_BATCH, 1), jnp.float32),
    )(
        g,
        par,
        W1,
        b1.reshape(1, _H1),
        W2,
        b2.reshape(1, _H2),
        Wout,
        bout.reshape(1, 1),
    )
    return out[:, 0]


# speedup vs baseline: 2.1657x; 1.8610x over previous
import functools

import jax
import jax.numpy as jnp
from jax.experimental import pallas as pl
from jax.experimental.pallas import tpu as pltpu
from jax.experimental.pallas import tpu_sc as plsc

_NUM_ENTITIES = 1000000
_EMBED = 64
_BATCH = 16384
_H1, _H2 = 512, 256
_FEAT = 7 * _EMBED

_NUM_IDX = 3 * _BATCH
_GATHER_WINDOW = 128
_BLOCK_B = 2048


_SC_CORES = 2
_SC_SUBCORES = 16
_NW = _SC_CORES * _SC_SUBCORES
_B_PER_W = _NUM_IDX // _NW


_WIN = 128
_WIDE = 2 * _EMBED
_HALF = _NUM_ENTITIES // 2
_PACK_R = 4096
_PACK_GRID = (_NUM_ENTITIES + 2 * _PACK_R - 1) // (2 * _PACK_R)
_TBLW_ROWS = _PACK_GRID * _PACK_R


def _pack_kernel(t_ref, o_ref):
    x = t_ref[...]
    o_ref[:, :_EMBED] = jnp.transpose(x[:, :_PACK_R])
    o_ref[:, _EMBED:] = jnp.transpose(x[:, _PACK_R:])


def _tc_pack(table_t):
    return pl.pallas_call(
        _pack_kernel,
        grid=(_PACK_GRID,),
        in_specs=[pl.BlockSpec((_EMBED, 2 * _PACK_R), lambda i: (0, i))],
        out_specs=pl.BlockSpec((_PACK_R, _WIDE), lambda i: (i, 0)),
        out_shape=jax.ShapeDtypeStruct((_TBLW_ROWS, _WIDE), jnp.float32),
        compiler_params=pltpu.CompilerParams(
            dimension_semantics=("parallel",)
        ),
    )(table_t)


def _sc_gather_wide(tablew, idxw):
    mesh = plsc.VectorSubcoreMesh(core_axis_name="c", subcore_axis_name="s")

    @functools.partial(
        pl.kernel,
        mesh=mesh,
        out_type=jax.ShapeDtypeStruct((_NUM_IDX, _WIDE), jnp.float32),
    )
    def gather_kernel(tbl_hbm, idx_hbm, out_hbm):
        def body(i_vmem, o_vmem):
            pltpu.sync_copy(tbl_hbm.at[i_vmem.at[0]], o_vmem)

        pltpu.emit_pipeline(
            body,
            grid=(_NUM_IDX // _WIN,),
            in_specs=[pl.BlockSpec((1, _WIN), lambda i: (0, i))],
            out_specs=[pl.BlockSpec((_WIN, _WIDE), lambda i: (i, 0))],
            core_axis_name=("c", "s"),
            dimension_semantics=(pltpu.PARALLEL,),
        )(idx_hbm, out_hbm)

    return gather_kernel(tablew, idxw.reshape(1, _NUM_IDX))


def _mlp_kernel(
    g_ref, p_ref, w1_ref, b1_ref, w2_ref, b2_ref, wout_ref, bout_ref, o_ref
):
    def pick(k):
        g = g_ref[k]
        p = p_ref[k]
        return jnp.where(p == 0, g[:, :_EMBED], g[:, _EMBED:])

    e1 = pick(0)
    e2 = pick(1)
    et = pick(2)
    feats = jnp.concatenate(
        [e1, e2, et, e1 * e2, e1 * et, e2 * et, e1 - e2], axis=-1
    )
    h = jnp.dot(feats, w1_ref[...], preferred_element_type=jnp.float32)
    h = jnp.maximum(h + b1_ref[...], 0.0)
    h = jnp.dot(h, w2_ref[...], preferred_element_type=jnp.float32)
    h = jnp.maximum(h + b2_ref[...], 0.0)
    out = jnp.dot(h, wout_ref[...], preferred_element_type=jnp.float32)
    o_ref[...] = out + bout_ref[0, 0]


def kernel(c1, c2, target, table, W1, b1, W2, b2, Wout, bout):
    idx = jnp.concatenate([c1, c2, target], axis=0).astype(jnp.int32)
    blk = idx // (2 * _PACK_R)
    rem = idx % (2 * _PACK_R)
    idxw = blk * _PACK_R + jnp.where(rem < _PACK_R, rem, rem - _PACK_R)
    par = (rem >= _PACK_R).astype(jnp.int32).reshape(3, _BATCH, 1)
    tablew = _tc_pack(jnp.transpose(table))
    g = _sc_gather_wide(tablew, idxw).reshape(3, _BATCH, _WIDE)

    out = pl.pallas_call(
        _mlp_kernel,
        grid=(_BATCH // _BLOCK_B,),
        in_specs=[
            pl.BlockSpec((3, _BLOCK_B, _WIDE), lambda i: (0, i, 0)),
            pl.BlockSpec((3, _BLOCK_B, 1), lambda i: (0, i, 0)),
            pl.BlockSpec((_FEAT, _H1), lambda i: (0, 0)),
            pl.BlockSpec((1, _H1), lambda i: (0, 0)),
            pl.BlockSpec((_H1, _H2), lambda i: (0, 0)),
            pl.BlockSpec((1, _H2), lambda i: (0, 0)),
            pl.BlockSpec((_H2, 1), lambda i: (0, 0)),
            pl.BlockSpec((1, 1), lambda i: (0, 0)),
        ],
        out_specs=pl.BlockSpec((_BLOCK_B, 1), lambda i: (i, 0)),
        out_shape=jax.ShapeDtypeStruct((_BATCH, 1), jnp.float32),
    )(
        g,
        par,
        W1,
        b1.reshape(1, _H1),
        W2,
        b2.reshape(1, _H2),
        Wout,
        bout.reshape(1, 1),
    )
    return out[:, 0]


# speedup vs baseline: 2.5646x; 1.1842x over previous
import functools

import jax
import jax.numpy as jnp
from jax.experimental import pallas as pl
from jax.experimental.pallas import tpu as pltpu
from jax.experimental.pallas import tpu_sc as plsc

_NUM_ENTITIES = 1000000
_EMBED = 64
_BATCH = 16384
_H1, _H2 = 512, 256
_FEAT = 7 * _EMBED

_NUM_IDX = 3 * _BATCH
_GATHER_WINDOW = 128
_BLOCK_B = 2048


_SC_CORES = 2
_SC_SUBCORES = 16
_NW = _SC_CORES * _SC_SUBCORES
_B_PER_W = _NUM_IDX // _NW


_WIN = 128
_WIDE = 2 * _EMBED
_HALF = _NUM_ENTITIES // 2
_PACK_R = 8192
_PACK_GRID = (_NUM_ENTITIES + 2 * _PACK_R - 1) // (2 * _PACK_R)
_TBLW_ROWS = _PACK_GRID * _PACK_R


_PACK_C = 2048


def _pack_kernel(t_ref, o_ref):
    for c in range(2 * _PACK_R // _PACK_C):
        y = jnp.transpose(t_ref[:, c * _PACK_C : (c + 1) * _PACK_C])
        half = (c * _PACK_C) // _PACK_R
        row = (c * _PACK_C) % _PACK_R
        o_ref[row : row + _PACK_C, half * _EMBED : (half + 1) * _EMBED] = y


def _tc_pack(table_t):
    return pl.pallas_call(
        _pack_kernel,
        grid=(_PACK_GRID,),
        in_specs=[pl.BlockSpec((_EMBED, 2 * _PACK_R), lambda i: (0, i))],
        out_specs=pl.BlockSpec((_PACK_R, _WIDE), lambda i: (i, 0)),
        out_shape=jax.ShapeDtypeStruct((_TBLW_ROWS, _WIDE), jnp.float32),
        compiler_params=pltpu.CompilerParams(
            dimension_semantics=("parallel",),
            vmem_limit_bytes=60 * 1024 * 1024,
        ),
    )(table_t)


def _sc_gather_wide(tablew, idxw):
    mesh = plsc.VectorSubcoreMesh(core_axis_name="c", subcore_axis_name="s")

    @functools.partial(
        pl.kernel,
        mesh=mesh,
        out_type=jax.ShapeDtypeStruct((_NUM_IDX, _WIDE), jnp.float32),
    )
    def gather_kernel(tbl_hbm, idx_hbm, out_hbm):
        def body(i_vmem, o_vmem):
            pltpu.sync_copy(tbl_hbm.at[i_vmem.at[0]], o_vmem)

        pltpu.emit_pipeline(
            body,
            grid=(_NUM_IDX // _WIN,),
            in_specs=[pl.BlockSpec((1, _WIN), lambda i: (0, i))],
            out_specs=[pl.BlockSpec((_WIN, _WIDE), lambda i: (i, 0))],
            core_axis_name=("c", "s"),
            dimension_semantics=(pltpu.PARALLEL,),
        )(idx_hbm, out_hbm)

    return gather_kernel(tablew, idxw.reshape(1, _NUM_IDX))


def _mlp_kernel(
    g_ref, i_ref, w1_ref, b1_ref, w2_ref, b2_ref, wout_ref, bout_ref, o_ref
):
    rem_t = jnp.transpose(i_ref[...]) % (2 * _PACK_R)

    def pick(k):
        g = g_ref[k]
        p = rem_t[:, k : k + 1] >= _PACK_R
        return jnp.where(p, g[:, _EMBED:], g[:, :_EMBED])

    e1 = pick(0)
    e2 = pick(1)
    et = pick(2)
    feats = jnp.concatenate(
        [e1, e2, et, e1 * e2, e1 * et, e2 * et, e1 - e2], axis=-1
    )
    h = jnp.dot(feats, w1_ref[...], preferred_element_type=jnp.float32)
    h = jnp.maximum(h + b1_ref[...], 0.0)
    h = jnp.dot(h, w2_ref[...], preferred_element_type=jnp.float32)
    h = jnp.maximum(h + b2_ref[...], 0.0)
    out = jnp.dot(h, wout_ref[...], preferred_element_type=jnp.float32)
    o_ref[...] = out + bout_ref[0, 0]


def kernel(c1, c2, target, table, W1, b1, W2, b2, Wout, bout):
    idx = jnp.concatenate([c1, c2, target], axis=0).astype(jnp.int32)
    blk = idx // (2 * _PACK_R)
    rem = idx % (2 * _PACK_R)
    idxw = blk * _PACK_R + jnp.where(rem < _PACK_R, rem, rem - _PACK_R)
    idx3 = idx.reshape(3, _BATCH)
    tablew = _tc_pack(jnp.transpose(table))
    g = _sc_gather_wide(tablew, idxw).reshape(3, _BATCH, _WIDE)

    out = pl.pallas_call(
        _mlp_kernel,
        grid=(_BATCH // _BLOCK_B,),
        in_specs=[
            pl.BlockSpec((3, _BLOCK_B, _WIDE), lambda i: (0, i, 0)),
            pl.BlockSpec((3, _BLOCK_B), lambda i: (0, i)),
            pl.BlockSpec((_FEAT, _H1), lambda i: (0, 0)),
            pl.BlockSpec((1, _H1), lambda i: (0, 0)),
            pl.BlockSpec((_H1, _H2), lambda i: (0, 0)),
            pl.BlockSpec((1, _H2), lambda i: (0, 0)),
            pl.BlockSpec((_H2, 1), lambda i: (0, 0)),
            pl.BlockSpec((1, 1), lambda i: (0, 0)),
        ],
        out_specs=pl.BlockSpec((_BLOCK_B, 1), lambda i: (i, 0)),
        out_shape=jax.ShapeDtypeStruct((_BATCH, 1), jnp.float32),
    )(
        g,
        idx3,
        W1,
        b1.reshape(1, _H1),
        W2,
        b2.reshape(1, _H2),
        Wout,
        bout.reshape(1, 1),
    )
    return out[:, 0]


# speedup vs baseline: 2.6960x; 1.0512x over previous
import functools

import jax
import jax.numpy as jnp
from jax.experimental import pallas as pl
from jax.experimental.pallas import tpu as pltpu
from jax.experimental.pallas import tpu_sc as plsc

_NUM_ENTITIES = 1000000
_EMBED = 64
_BATCH = 16384
_H1, _H2 = 512, 256
_FEAT = 7 * _EMBED

_NUM_IDX = 3 * _BATCH
_GATHER_WINDOW = 128
_BLOCK_B = 4096


_SC_CORES = 2
_SC_SUBCORES = 16
_NW = _SC_CORES * _SC_SUBCORES
_B_PER_W = _NUM_IDX // _NW


_WIN = 128
_WIDE = 2 * _EMBED
_HALF = _NUM_ENTITIES // 2
_PACK_R = 16384
_PACK_GRID = (_NUM_ENTITIES + 2 * _PACK_R - 1) // (2 * _PACK_R)
_TBLW_ROWS = _PACK_GRID * _PACK_R


_PACK_C = 2048


def _pack_kernel(t_ref, o_ref):
    for c in range(2 * _PACK_R // _PACK_C):
        y = jnp.transpose(t_ref[:, c * _PACK_C : (c + 1) * _PACK_C])
        half = (c * _PACK_C) // _PACK_R
        row = (c * _PACK_C) % _PACK_R
        o_ref[row : row + _PACK_C, half * _EMBED : (half + 1) * _EMBED] = y


def _tc_pack(table_t):
    return pl.pallas_call(
        _pack_kernel,
        grid=(_PACK_GRID,),
        in_specs=[pl.BlockSpec((_EMBED, 2 * _PACK_R), lambda i: (0, i))],
        out_specs=pl.BlockSpec((_PACK_R, _WIDE), lambda i: (i, 0)),
        out_shape=jax.ShapeDtypeStruct((_TBLW_ROWS, _WIDE), jnp.float32),
        compiler_params=pltpu.CompilerParams(
            dimension_semantics=("parallel",),
            vmem_limit_bytes=60 * 1024 * 1024,
        ),
    )(table_t)


def _sc_gather_wide(tablew, idxw):
    mesh = plsc.VectorSubcoreMesh(core_axis_name="c", subcore_axis_name="s")

    @functools.partial(
        pl.kernel,
        mesh=mesh,
        out_type=jax.ShapeDtypeStruct((_NUM_IDX, _WIDE), jnp.float32),
    )
    def gather_kernel(tbl_hbm, idx_hbm, out_hbm):
        def body(i_vmem, o_vmem):
            pltpu.sync_copy(tbl_hbm.at[i_vmem.at[0]], o_vmem)

        pltpu.emit_pipeline(
            body,
            grid=(_NUM_IDX // _WIN,),
            in_specs=[pl.BlockSpec((1, _WIN), lambda i: (0, i))],
            out_specs=[pl.BlockSpec((_WIN, _WIDE), lambda i: (i, 0))],
            core_axis_name=("c", "s"),
            dimension_semantics=(pltpu.PARALLEL,),
        )(idx_hbm, out_hbm)

    return gather_kernel(tablew, idxw.reshape(1, _NUM_IDX))


def _mlp_kernel(
    g_ref, i_ref, w1_ref, b1_ref, w2_ref, b2_ref, wout_ref, bout_ref, o_ref
):
    rem_t = jnp.transpose(i_ref[...]) % (2 * _PACK_R)

    def pick(k):
        g = g_ref[k]
        p = rem_t[:, k : k + 1] >= _PACK_R
        return jnp.where(p, g[:, _EMBED:], g[:, :_EMBED])

    e1 = pick(0)
    e2 = pick(1)
    et = pick(2)
    feats = jnp.concatenate(
        [e1, e2, et, e1 * e2, e1 * et, e2 * et, e1 - e2], axis=-1
    )
    h = jnp.dot(feats, w1_ref[...], preferred_element_type=jnp.float32)
    h = jnp.maximum(h + b1_ref[...], 0.0)
    h = jnp.dot(h, w2_ref[...], preferred_element_type=jnp.float32)
    h = jnp.maximum(h + b2_ref[...], 0.0)
    out = jnp.dot(h, wout_ref[...], preferred_element_type=jnp.float32)
    o_ref[...] = out + bout_ref[0, 0]


def kernel(c1, c2, target, table, W1, b1, W2, b2, Wout, bout):
    idx = jnp.concatenate([c1, c2, target], axis=0).astype(jnp.int32)
    blk = idx // (2 * _PACK_R)
    rem = idx % (2 * _PACK_R)
    idxw = blk * _PACK_R + jnp.where(rem < _PACK_R, rem, rem - _PACK_R)
    idx3 = idx.reshape(3, _BATCH)
    tablew = _tc_pack(jnp.transpose(table))
    g = _sc_gather_wide(tablew, idxw).reshape(3, _BATCH, _WIDE)

    out = pl.pallas_call(
        _mlp_kernel,
        grid=(_BATCH // _BLOCK_B,),
        in_specs=[
            pl.BlockSpec((3, _BLOCK_B, _WIDE), lambda i: (0, i, 0)),
            pl.BlockSpec((3, _BLOCK_B), lambda i: (0, i)),
            pl.BlockSpec((_FEAT, _H1), lambda i: (0, 0)),
            pl.BlockSpec((1, _H1), lambda i: (0, 0)),
            pl.BlockSpec((_H1, _H2), lambda i: (0, 0)),
            pl.BlockSpec((1, _H2), lambda i: (0, 0)),
            pl.BlockSpec((_H2, 1), lambda i: (0, 0)),
            pl.BlockSpec((1, 1), lambda i: (0, 0)),
        ],
        out_specs=pl.BlockSpec((_BLOCK_B, 1), lambda i: (i, 0)),
        out_shape=jax.ShapeDtypeStruct((_BATCH, 1), jnp.float32),
    )(
        g,
        idx3,
        W1,
        b1.reshape(1, _H1),
        W2,
        b2.reshape(1, _H2),
        Wout,
        bout.reshape(1, 1),
    )
    return out[:, 0]
